# baseline (speedup 1.0000x reference)
import jax
import jax.numpy as jnp
from jax.experimental import pallas as pl
from jax.experimental.pallas import tpu as pltpu


def _mix_kernel(x_ref, wt_ref, o_ref):
    y = jax.lax.dot_general(
        wt_ref[...], x_ref[0],
        (((1,), (0,)), ((), ())),
        preferred_element_type=jnp.float32,
    )
    o_ref[0, 0] = y
    o_ref[0, 1] = y
    o_ref[0, 2] = y
    o_ref[0, 3] = y


def kernel(x, tind0, tind1, pind0, pind1, W):
    b, d, h, w = x.shape
    l = h * w
    k = 4
    xf = x.reshape(b, d, l)
    wt = W.T

    n_e = 2
    eb = d // n_e
    return pl.pallas_call(
        _mix_kernel,
        grid=(b, n_e),
        in_specs=[
            pl.BlockSpec((1, d, l), lambda i, j: (i, 0, 0)),
            pl.BlockSpec((eb, d), lambda i, j: (j, 0)),
        ],
        out_specs=pl.BlockSpec((1, k, eb, l), lambda i, j: (i, 0, j, 0)),
        out_shape=jax.ShapeDtypeStruct((b, k, d, l), jnp.float32),
        compiler_params=pltpu.CompilerParams(
            dimension_semantics=("parallel", "arbitrary"),
        ),
    )(xf, wt)


# speedup vs baseline: 1.1173x; 1.1173x over previous
import jax
import jax.numpy as jnp
from jax.experimental import pallas as pl
from jax.experimental.pallas import tpu as pltpu


def _mix_kernel(x_ref, w_ref, o_ref):
    y = jax.lax.dot_general(
        w_ref[...], x_ref[0],
        (((0,), (0,)), ((), ())),
        preferred_element_type=jnp.float32,
    )
    o_ref[0, 0] = y
    o_ref[0, 1] = y
    o_ref[0, 2] = y
    o_ref[0, 3] = y


def kernel(x, tind0, tind1, pind0, pind1, W):
    b, d, h, w = x.shape
    l = h * w
    k = 4
    xf = x.reshape(b, d, l)

    return pl.pallas_call(
        _mix_kernel,
        grid=(b,),
        in_specs=[
            pl.BlockSpec((1, d, l), lambda i: (i, 0, 0)),
            pl.BlockSpec((d, d), lambda i: (0, 0)),
        ],
        out_specs=pl.BlockSpec((1, k, d, l), lambda i: (i, 0, 0, 0)),
        out_shape=jax.ShapeDtypeStruct((b, k, d, l), jnp.float32),
        compiler_params=pltpu.CompilerParams(
            dimension_semantics=("parallel",),
        ),
    )(xf, W)


# speedup vs baseline: 1.1173x; 1.0001x over previous
import jax
import jax.numpy as jnp
from jax.experimental import pallas as pl
from jax.experimental.pallas import tpu as pltpu


def _mix_kernel(x_ref, w_ref, o_ref):
    for bi in range(x_ref.shape[0]):
        y = jax.lax.dot_general(
            w_ref[...], x_ref[bi],
            (((0,), (0,)), ((), ())),
            preferred_element_type=jnp.float32,
        )
        o_ref[bi, 0] = y
        o_ref[bi, 1] = y
        o_ref[bi, 2] = y
        o_ref[bi, 3] = y


def kernel(x, tind0, tind1, pind0, pind1, W):
    b, d, h, w = x.shape
    l = h * w
    k = 4
    xf = x.reshape(b, d, l)

    bb = 2
    return pl.pallas_call(
        _mix_kernel,
        grid=(b // bb,),
        in_specs=[
            pl.BlockSpec((bb, d, l), lambda i: (i, 0, 0)),
            pl.BlockSpec((d, d), lambda i: (0, 0)),
        ],
        out_specs=pl.BlockSpec((bb, k, d, l), lambda i: (i, 0, 0, 0)),
        out_shape=jax.ShapeDtypeStruct((b, k, d, l), jnp.float32),
        compiler_params=pltpu.CompilerParams(
            dimension_semantics=("parallel",),
        ),
    )(xf, W)


# speedup vs baseline: 1.1198x; 1.0022x over previous
import jax
import jax.numpy as jnp
from jax.experimental import pallas as pl
from jax.experimental.pallas import tpu as pltpu


def _mix_kernel(x_ref, w_ref, o_ref):
    y = jax.lax.dot_general(
        w_ref[...], x_ref[0],
        (((0,), (0,)), ((), ())),
        preferred_element_type=jnp.float32,
    )
    o_ref[0, 0] = y
    o_ref[0, 1] = y
    o_ref[0, 2] = y
    o_ref[0, 3] = y


def kernel(x, tind0, tind1, pind0, pind1, W):
    b, d, h, w = x.shape
    l = h * w
    k = 4
    xf = x.reshape(b, d, l)

    return pl.pallas_call(
        _mix_kernel,
        grid=(b,),
        in_specs=[
            pl.BlockSpec((1, d, l), lambda i: (i, 0, 0)),
            pl.BlockSpec((d, d), lambda i: (0, 0)),
        ],
        out_specs=pl.BlockSpec((1, k, d, l), lambda i: (i, 0, 0, 0)),
        out_shape=jax.ShapeDtypeStruct((b, k, d, l), jnp.float32),
        compiler_params=pltpu.CompilerParams(
            dimension_semantics=("parallel",),
        ),
    )(xf, W)
